# baseline (speedup 1.0000x reference)
import functools

import jax
import jax.numpy as jnp
from jax import lax
from jax.experimental import pallas as pl
from jax.experimental.pallas import tpu as pltpu
from jax.experimental.pallas import tpu_sc as plsc

N_IN, N_OUT, K, C_IN, C_OUT, HIDDEN, BATCH = 10000, 1000, 16, 128, 32, 32, 4

_NW = 32
_D = BATCH * C_IN
_ROWS = N_OUT * K
_L1, _L2 = 520, 480
_NCH = 4
_CH1, _CH2 = 72, 64


def _sc_gather(table, idx3, chunk):
    mesh = plsc.VectorSubcoreMesh(core_axis_name="c", subcore_axis_name="s")
    rows_per_w = _NCH * chunk

    @functools.partial(
        pl.kernel,
        mesh=mesh,
        out_type=jax.ShapeDtypeStruct((_NW * rows_per_w, _D), jnp.float32),
        scratch_types=[
            pltpu.VMEM((_NCH, chunk), jnp.int32),
            pltpu.VMEM((chunk, _D), jnp.float32),
            pltpu.VMEM((chunk, _D), jnp.float32),
            pltpu.SemaphoreType.DMA,
            pltpu.SemaphoreType.DMA,
            pltpu.SemaphoreType.DMA,
            pltpu.SemaphoreType.DMA,
        ],
    )
    def k(table_hbm, idx_hbm, out_hbm, idx_v, rows0, rows1, sg0, sg1, sw0, sw1):
        wid = lax.axis_index("s") * 2 + lax.axis_index("c")
        base = wid * rows_per_w
        pltpu.sync_copy(idx_hbm.at[wid], idx_v)
        bufs, gsems, wsems = (rows0, rows1), (sg0, sg1), (sw0, sw1)

        gh = [pltpu.async_copy(table_hbm.at[idx_v.at[g]], bufs[g], gsems[g])
              for g in range(2)]
        wh = [None, None]
        for g in range(_NCH):
            b = g & 1
            gh[b].wait()
            wh[b] = pltpu.async_copy(
                bufs[b], out_hbm.at[pl.ds(base + g * chunk, chunk)], wsems[b])
            if g + 2 < _NCH:
                wh[b].wait()
                gh[b] = pltpu.async_copy(
                    table_hbm.at[idx_v.at[g + 2]], bufs[b], gsems[b])
        wh[0].wait()
        wh[1].wait()

    return k(table, idx3)


_LB = 40
_EB = _LB * K
_MR = HIDDEN * _LB


def _tc_body(cx_ref, cy_ref, xg_ref, wp_ref, e16_ref, mask_ref, wbd_ref,
             b2bd_ref, out_ref):
    f32, bf16 = jnp.float32, jnp.bfloat16
    cx = cx_ref[...].astype(bf16).astype(f32)
    cy = cy_ref[...].astype(bf16).astype(f32)
    w1x = wp_ref[0, :HIDDEN].astype(bf16).astype(f32)
    w1y = wp_ref[1, :HIDDEN].astype(bf16).astype(f32)
    arg = (cx[:, :, None] * w1x[None, None, :]
           + cy[:, :, None] * w1y[None, None, :]) + wp_ref[2, :HIDDEN][None, None, :]
    h = jnp.sin(30.0 * arg)
    ht = jnp.transpose(h, (2, 0, 1)).reshape(_MR, K)
    hbd = jnp.dot(ht, e16_ref[...],
                  preferred_element_type=f32) * mask_ref[...]
    xg = xg_ref[...].reshape(_EB, _D)
    p = jnp.dot(hbd, xg, preferred_element_type=f32)
    sx = jnp.sum(xg_ref[...], axis=1)
    res = (jnp.dot(sx, b2bd_ref[...], preferred_element_type=f32)
           + wp_ref[3, :][None, :])
    for m in range(HIDDEN):
        res = res + jnp.dot(p[m * _LB:(m + 1) * _LB, :], wbd_ref[m],
                            preferred_element_type=f32)
    out_ref[...] = res


def _tc_call(cx, cy, xg3, wp, e16, mask, wbd, b2bd):
    n_half = cx.shape[0]
    grid = (n_half // _LB,)
    return pl.pallas_call(
        _tc_body,
        grid=grid,
        in_specs=[
            pl.BlockSpec((_LB, K), lambda i: (i, 0)),
            pl.BlockSpec((_LB, K), lambda i: (i, 0)),
            pl.BlockSpec((_LB, K, _D), lambda i: (i, 0, 0)),
            pl.BlockSpec((8, BATCH * C_OUT), lambda i: (0, 0)),
            pl.BlockSpec((K, _EB), lambda i: (0, 0)),
            pl.BlockSpec((_MR, _EB), lambda i: (0, 0)),
            pl.BlockSpec((HIDDEN, _D, BATCH * C_OUT), lambda i: (0, 0, 0)),
            pl.BlockSpec((_D, BATCH * C_OUT), lambda i: (0, 0)),
        ],
        out_specs=pl.BlockSpec((_LB, BATCH * C_OUT), lambda i: (i, 0)),
        out_shape=jax.ShapeDtypeStruct((n_half, BATCH * C_OUT), jnp.float32),
    )(cx, cy, xg3, wp, e16, mask, wbd, b2bd)


def kernel(x, locs_unfold, W1, b1, W2, b2, bias, neighbours):
    f32 = jnp.float32
    lu3 = locs_unfold.reshape(2, K, N_OUT)
    cx = lu3[0].T
    cy = lu3[1].T
    wp = jnp.zeros((8, BATCH * C_OUT), f32)
    wp = wp.at[0, :HIDDEN].set(W1[0]).at[1, :HIDDEN].set(W1[1])
    wp = wp.at[2, :HIDDEN].set(b1)
    wp = wp.at[3].set(jnp.tile(bias[0, :, 0], BATCH))
    w2q3 = W2.reshape(HIDDEN, C_OUT, C_IN).transpose(0, 2, 1)
    b2r = b2.reshape(C_OUT, C_IN).T
    eyeb = jnp.eye(BATCH, dtype=f32)
    wbd = (w2q3[:, None, :, None, :] * eyeb[None, :, None, :, None]).reshape(
        HIDDEN, _D, BATCH * C_OUT)
    b2bd = (b2r[None, :, None, :] * eyeb[:, None, :, None]).reshape(
        _D, BATCH * C_OUT)
    e16 = jnp.tile(jnp.eye(K, dtype=f32), (1, _LB))
    rowl = jnp.arange(_MR, dtype=jnp.int32) % _LB
    coll = jnp.arange(_EB, dtype=jnp.int32) // K
    mask = (rowl[:, None] == coll[None, :]).astype(f32)

    table = x.transpose(2, 0, 1).reshape(N_IN, BATCH * C_IN)
    idx = neighbours.T.reshape(_ROWS).astype(jnp.int32)
    r1, p1 = _L1 * K, _NW * _NCH * _CH1
    r2, p2 = _L2 * K, _NW * _NCH * _CH2
    pad1 = jnp.arange(p1 - r1, dtype=jnp.int32)
    pad2 = jnp.arange(p2 - r2, dtype=jnp.int32)
    idx1 = jnp.concatenate([idx[:r1], pad1]).reshape(_NW, _NCH, _CH1)
    idx2 = jnp.concatenate([idx[r1:], pad2]).reshape(_NW, _NCH, _CH2)
    xg1 = _sc_gather(table, idx1, _CH1)[:r1]
    xg2 = _sc_gather(table, idx2, _CH2)[:r2]

    o1 = _tc_call(cx[:_L1], cy[:_L1], xg1.reshape(_L1, K, _D),
                  wp, e16, mask, wbd, b2bd)
    o2 = _tc_call(cx[_L1:], cy[_L1:], xg2.reshape(_L2, K, _D),
                  wp, e16, mask, wbd, b2bd)
    out2 = jnp.concatenate([o1, o2], axis=0)
    return out2.reshape(N_OUT, BATCH, C_OUT).transpose(1, 2, 0)


# speedup vs baseline: 1.0404x; 1.0404x over previous
import functools

import jax
import jax.numpy as jnp
from jax import lax
from jax.experimental import pallas as pl
from jax.experimental.pallas import tpu as pltpu
from jax.experimental.pallas import tpu_sc as plsc

N_IN, N_OUT, K, C_IN, C_OUT, HIDDEN, BATCH = 10000, 1000, 16, 128, 32, 32, 4

_NW = 32
_D = BATCH * C_IN
_ROWS = N_OUT * K
_ROWS_PAD = 16384
_NCH = 8
_CHUNK = 64


def _sc_gather(table, idx3, chunk):
    mesh = plsc.VectorSubcoreMesh(core_axis_name="c", subcore_axis_name="s")
    rows_per_w = _NCH * chunk

    @functools.partial(
        pl.kernel,
        mesh=mesh,
        out_type=jax.ShapeDtypeStruct((_NW * rows_per_w, _D), jnp.float32),
        scratch_types=[
            pltpu.VMEM((_NCH, chunk), jnp.int32),
            pltpu.VMEM((chunk, _D), jnp.float32),
            pltpu.VMEM((chunk, _D), jnp.float32),
            pltpu.SemaphoreType.DMA,
            pltpu.SemaphoreType.DMA,
            pltpu.SemaphoreType.DMA,
            pltpu.SemaphoreType.DMA,
        ],
    )
    def k(table_hbm, idx_hbm, out_hbm, idx_v, rows0, rows1, sg0, sg1, sw0, sw1):
        wid = lax.axis_index("s") * 2 + lax.axis_index("c")
        base = wid * rows_per_w
        pltpu.sync_copy(idx_hbm.at[wid], idx_v)
        bufs, gsems, wsems = (rows0, rows1), (sg0, sg1), (sw0, sw1)

        gh = [pltpu.async_copy(table_hbm.at[idx_v.at[g]], bufs[g], gsems[g])
              for g in range(2)]
        wh = [None, None]
        for g in range(_NCH):
            b = g & 1
            gh[b].wait()
            wh[b] = pltpu.async_copy(
                bufs[b], out_hbm.at[pl.ds(base + g * chunk, chunk)], wsems[b])
            if g + 2 < _NCH:
                wh[b].wait()
                gh[b] = pltpu.async_copy(
                    table_hbm.at[idx_v.at[g + 2]], bufs[b], gsems[b])
        wh[0].wait()
        wh[1].wait()

    return k(table, idx3)


_LB = 40
_EB = _LB * K
_MR = HIDDEN * _LB


def _tc_body(cx_ref, cy_ref, xg_ref, wp_ref, e16_ref, mask_ref, wbd_ref,
             b2bd_ref, out_ref):
    f32, bf16 = jnp.float32, jnp.bfloat16
    cx = cx_ref[...].astype(bf16).astype(f32)
    cy = cy_ref[...].astype(bf16).astype(f32)
    w1x = wp_ref[0, :HIDDEN].astype(bf16).astype(f32)
    w1y = wp_ref[1, :HIDDEN].astype(bf16).astype(f32)
    arg = (cx[:, :, None] * w1x[None, None, :]
           + cy[:, :, None] * w1y[None, None, :]) + wp_ref[2, :HIDDEN][None, None, :]
    h = jnp.sin(30.0 * arg)
    ht = jnp.transpose(h, (2, 0, 1)).reshape(_MR, K)
    hbd = jnp.dot(ht, e16_ref[...],
                  preferred_element_type=f32) * mask_ref[...]
    xg = xg_ref[...].reshape(_EB, _D)
    p = jnp.dot(hbd, xg, preferred_element_type=f32)
    sx = jnp.sum(xg_ref[...], axis=1)
    res = (jnp.dot(sx, b2bd_ref[...], preferred_element_type=f32)
           + wp_ref[3, :][None, :])
    for m in range(HIDDEN):
        res = res + jnp.dot(p[m * _LB:(m + 1) * _LB, :], wbd_ref[m],
                            preferred_element_type=f32)
    out_ref[...] = res


def _tc_call(cx, cy, xg3, wp, e16, mask, wbd, b2bd):
    n_half = cx.shape[0]
    grid = (n_half // _LB,)
    return pl.pallas_call(
        _tc_body,
        grid=grid,
        in_specs=[
            pl.BlockSpec((_LB, K), lambda i: (i, 0)),
            pl.BlockSpec((_LB, K), lambda i: (i, 0)),
            pl.BlockSpec((_LB, K, _D), lambda i: (i, 0, 0)),
            pl.BlockSpec((8, BATCH * C_OUT), lambda i: (0, 0)),
            pl.BlockSpec((K, _EB), lambda i: (0, 0)),
            pl.BlockSpec((_MR, _EB), lambda i: (0, 0)),
            pl.BlockSpec((HIDDEN, _D, BATCH * C_OUT), lambda i: (0, 0, 0)),
            pl.BlockSpec((_D, BATCH * C_OUT), lambda i: (0, 0)),
        ],
        out_specs=pl.BlockSpec((_LB, BATCH * C_OUT), lambda i: (i, 0)),
        out_shape=jax.ShapeDtypeStruct((n_half, BATCH * C_OUT), jnp.float32),
    )(cx, cy, xg3, wp, e16, mask, wbd, b2bd)


def kernel(x, locs_unfold, W1, b1, W2, b2, bias, neighbours):
    f32 = jnp.float32
    lu3 = locs_unfold.reshape(2, K, N_OUT)
    cx = lu3[0].T
    cy = lu3[1].T
    wp = jnp.zeros((8, BATCH * C_OUT), f32)
    wp = wp.at[0, :HIDDEN].set(W1[0]).at[1, :HIDDEN].set(W1[1])
    wp = wp.at[2, :HIDDEN].set(b1)
    wp = wp.at[3].set(jnp.tile(bias[0, :, 0], BATCH))
    w2q3 = W2.reshape(HIDDEN, C_OUT, C_IN).transpose(0, 2, 1)
    b2r = b2.reshape(C_OUT, C_IN).T
    eyeb = jnp.eye(BATCH, dtype=f32)
    wbd = (w2q3[:, None, :, None, :] * eyeb[None, :, None, :, None]).reshape(
        HIDDEN, _D, BATCH * C_OUT)
    b2bd = (b2r[None, :, None, :] * eyeb[:, None, :, None]).reshape(
        _D, BATCH * C_OUT)
    e16 = jnp.tile(jnp.eye(K, dtype=f32), (1, _LB))
    rowl = jnp.arange(_MR, dtype=jnp.int32) % _LB
    coll = jnp.arange(_EB, dtype=jnp.int32) // K
    mask = (rowl[:, None] == coll[None, :]).astype(f32)

    table = x.transpose(2, 0, 1).reshape(N_IN, BATCH * C_IN)
    pad = jnp.arange(_ROWS_PAD - _ROWS, dtype=jnp.int32)
    idx_pad = jnp.concatenate(
        [neighbours.T.reshape(_ROWS).astype(jnp.int32), pad])
    xg = _sc_gather(table, idx_pad.reshape(_NW, _NCH, _CHUNK), _CHUNK)
    xg3 = xg[:_ROWS].reshape(N_OUT, K, _D)

    out2 = _tc_call(cx, cy, xg3, wp, e16, mask, wbd, b2bd)
    return out2.reshape(N_OUT, BATCH, C_OUT).transpose(1, 2, 0)
